# baseline (speedup 1.0000x reference)
import functools

import jax
import jax.numpy as jnp
from jax import lax
from jax.experimental import pallas as pl
from jax.experimental.pallas import tpu as pltpu
from jax.experimental.pallas import tpu_sc as plsc

VOCAB_SIZE = 170000
EMB_D = 16
BATCH_N = 16384
SEQ_N = 200

TC_ROWS = VOCAB_SIZE * EMB_D // 128

NUM_CORES = 2
NUM_SUBCORES = 16
NUM_WORKERS = NUM_CORES * NUM_SUBCORES
ROWS_PER_WORKER = BATCH_N // NUM_WORKERS
CHUNK_ROWS = 64
NUM_CHUNKS = ROWS_PER_WORKER // CHUNK_ROWS
CHUNK_ELEMS = CHUNK_ROWS * SEQ_N
NUM_PAIRS = VOCAB_SIZE // 2
LANES = 16
GROUPS_PER_CHUNK = CHUNK_ROWS // LANES
S_UNROLL = 8


TC_BLOCK = 5000
TC_GRID = NUM_PAIRS // TC_BLOCK
TC_SUB = 40
TC_MINOR = TC_BLOCK // TC_SUB


def _bf16_rn_bits(x):
    xb = lax.bitcast_convert_type(x, jnp.int32)
    odd = lax.bitwise_and(lax.shift_right_logical(xb, 16), jnp.int32(1))
    return lax.shift_right_logical(xb + jnp.int32(0x7FFF) + odd, 16)


def _tc_tw_body(tlo_ref, thi_ref, w_ref, b_ref, out_ref):
    w3 = w_ref[...].reshape(1, 1, EMB_D)
    lo = (
        jnp.sum(tlo_ref[...].reshape(TC_SUB, TC_MINOR, EMB_D) * w3, axis=2)
        + b_ref[0, 0]
    )
    hi = (
        jnp.sum(thi_ref[...].reshape(TC_SUB, TC_MINOR, EMB_D) * w3, axis=2)
        + b_ref[0, 0]
    )
    out_ref[...] = lax.bitwise_or(
        _bf16_rn_bits(lo), lax.shift_left(_bf16_rn_bits(hi), 16)
    )


def _compute_packed_tw(table, W, b):
    packed2 = pl.pallas_call(
        _tc_tw_body,
        grid=(TC_GRID,),
        out_shape=jax.ShapeDtypeStruct((TC_GRID * TC_SUB, TC_MINOR), jnp.int32),
        in_specs=[
            pl.BlockSpec((TC_BLOCK, EMB_D), lambda g: (g, 0)),
            pl.BlockSpec((TC_BLOCK, EMB_D), lambda g: (g + TC_GRID, 0)),
            pl.BlockSpec((1, EMB_D), lambda g: (0, 0)),
            pl.BlockSpec(memory_space=pltpu.SMEM),
        ],
        out_specs=pl.BlockSpec((TC_SUB, TC_MINOR), lambda g: (g, 0)),
    )(table, table, W.astype(jnp.float32), b.reshape(1, 1).astype(jnp.float32))
    return packed2.reshape(NUM_PAIRS)


def _sc_body(tw_hbm, x_hbm, out_hbm, twbuf, xb0, xb1, outbuf, sem_tw, sem_x0, sem_x1):
    wid = lax.axis_index("s") * NUM_CORES + lax.axis_index("c")
    row_base = wid * ROWS_PER_WORKER

    cp_tw = pltpu.async_copy(tw_hbm, twbuf, sem_tw)
    xbufs = (xb0, xb1)
    sems = (sem_x0, sem_x1)
    copies = [None, None]
    copies[0] = pltpu.async_copy(
        x_hbm.at[pl.ds(row_base, CHUNK_ROWS), :], xbufs[0], sems[0]
    )
    cp_tw.wait()

    lane = lax.iota(jnp.int32, LANES)
    hi_mask = jnp.int32(-65536)

    for c in range(NUM_CHUNKS):
        cur = c % 2
        nxt = (c + 1) % 2
        if c + 1 < NUM_CHUNKS:
            copies[nxt] = pltpu.async_copy(
                x_hbm.at[pl.ds(row_base + (c + 1) * CHUNK_ROWS, CHUNK_ROWS), :],
                xbufs[nxt],
                sems[nxt],
            )
        copies[cur].wait()
        xb = xbufs[cur]
        for g in range(GROUPS_PER_CHUNK):
            row_idx = g * LANES + lane

            def s_step(i, acc, row_idx=row_idx, xb=xb):
                s0 = i * S_UNROLL
                for k in range(S_UNROLL):
                    col = jnp.full((LANES,), s0 + k, jnp.int32)
                    tok = plsc.load_gather(xb, [row_idx, col])
                    in_hi = tok >= jnp.int32(NUM_PAIRS)
                    pidx = tok - jnp.where(in_hi, jnp.int32(NUM_PAIRS), jnp.int32(0))
                    pk = plsc.load_gather(twbuf, [pidx])
                    bits = jnp.where(
                        in_hi, lax.bitwise_and(pk, hi_mask), lax.shift_left(pk, 16)
                    )
                    acc = acc + plsc.bitcast(bits, jnp.float32)
                return acc

            acc = lax.fori_loop(
                0, SEQ_N // S_UNROLL, s_step, jnp.zeros((LANES,), jnp.float32)
            )
            z = acc * jnp.float32(1.0 / SEQ_N)
            res = 1.0 / (1.0 + jnp.exp(-z))
            outbuf[pl.ds(c * CHUNK_ROWS + g * LANES, LANES)] = res

    pltpu.sync_copy(outbuf, out_hbm.at[pl.ds(row_base, ROWS_PER_WORKER)])


@jax.jit
def kernel(x, table, W, b):
    packed = _compute_packed_tw(table, W, b)
    x2d = x.astype(jnp.int32)

    mesh = plsc.VectorSubcoreMesh(
        core_axis_name="c",
        subcore_axis_name="s",
        num_cores=NUM_CORES,
        num_subcores=NUM_SUBCORES,
    )
    out1d = pl.kernel(
        _sc_body,
        out_type=jax.ShapeDtypeStruct((BATCH_N,), jnp.float32),
        mesh=mesh,
        compiler_params=pltpu.CompilerParams(needs_layout_passes=False),
        scratch_types=[
            pltpu.VMEM((NUM_PAIRS,), jnp.int32),
            pltpu.VMEM((CHUNK_ROWS, SEQ_N), jnp.int32),
            pltpu.VMEM((CHUNK_ROWS, SEQ_N), jnp.int32),
            pltpu.VMEM((ROWS_PER_WORKER,), jnp.float32),
            pltpu.SemaphoreType.DMA,
            pltpu.SemaphoreType.DMA,
            pltpu.SemaphoreType.DMA,
        ],
    )(packed, x2d)
    return out1d.reshape(BATCH_N, 1)


# speedup vs baseline: 1.4350x; 1.4350x over previous
import jax
import jax.numpy as jnp
from jax import lax
from jax.experimental import pallas as pl
from jax.experimental.pallas import tpu as pltpu
from jax.experimental.pallas import tpu_sc as plsc

VOCAB_SIZE = 170000
EMB_D = 16
BATCH_N = 16384
SEQ_N = 200

NUM_CORES = 2
NUM_SUBCORES = 16
NUM_WORKERS = NUM_CORES * NUM_SUBCORES
ROWS_PER_WORKER = BATCH_N // NUM_WORKERS
CHUNK_ROWS = 64
NUM_CHUNKS = ROWS_PER_WORKER // CHUNK_ROWS
CHUNK_ELEMS = CHUNK_ROWS * SEQ_N
LANES = 16
GROUPS_PER_CHUNK = CHUNK_ROWS // LANES
S_UNROLL = 8

TC_ROWS_PER_BLOCK = 10240
TC_GRID = 17
TC_CHUNKS = TC_ROWS_PER_BLOCK // 128
TC_OUT_ROWS = TC_CHUNKS // 2
PACKED_WORDS = TC_GRID * TC_OUT_ROWS * 128


def _bf16_rn_bits(x):
    xb = lax.bitcast_convert_type(x, jnp.int32)
    odd = lax.bitwise_and(lax.shift_right_logical(xb, 16), jnp.int32(1))
    return lax.shift_right_logical(xb + jnp.int32(0x7FFF) + odd, 16)


def _tc_tw_body(tbl_ref, wb_ref, b_ref, out_ref):
    rep = jnp.dot(tbl_ref[...], wb_ref[...], preferred_element_type=jnp.float32)
    t = jnp.transpose(rep.reshape(TC_CHUNKS, 128, 128), (0, 2, 1))
    c = t[:, 0, :] + b_ref[0, 0]
    bits = _bf16_rn_bits(c)
    rolled = jnp.concatenate([bits[:, 64:], bits[:, :64]], axis=1)
    packed = lax.bitwise_or(bits, lax.shift_left(rolled, 16))
    p2 = packed.reshape(TC_OUT_ROWS, 2, 128)
    out_ref[...] = jnp.concatenate([p2[:, 0, :64], p2[:, 1, :64]], axis=1)


def _compute_packed_tw(table, W, b):
    wb = jnp.broadcast_to(W.reshape(EMB_D, 1).astype(jnp.float32), (EMB_D, 128))
    packed2 = pl.pallas_call(
        _tc_tw_body,
        grid=(TC_GRID,),
        out_shape=jax.ShapeDtypeStruct((TC_GRID * TC_OUT_ROWS, 128), jnp.int32),
        in_specs=[
            pl.BlockSpec((TC_ROWS_PER_BLOCK, EMB_D), lambda g: (g, 0)),
            pl.BlockSpec((EMB_D, 128), lambda g: (0, 0)),
            pl.BlockSpec(memory_space=pltpu.SMEM),
        ],
        out_specs=pl.BlockSpec((TC_OUT_ROWS, 128), lambda g: (g, 0)),
    )(table, wb, b.reshape(1, 1).astype(jnp.float32))
    return packed2.reshape(PACKED_WORDS)


def _sc_body(tw_hbm, x_hbm, out_hbm, twbuf, xb0, xb1, outbuf, sem_tw, sem_x0, sem_x1):
    wid = lax.axis_index("s") * NUM_CORES + lax.axis_index("c")
    row_base = wid * ROWS_PER_WORKER

    cp_tw = pltpu.async_copy(tw_hbm, twbuf, sem_tw)
    xbufs = (xb0, xb1)
    sems = (sem_x0, sem_x1)

    def fire_chunk(c_idx, buf_i):
        r0 = row_base + c_idx * CHUNK_ROWS
        return [
            pltpu.async_copy(
                x_hbm.at[pl.ds(r0, CHUNK_ROWS), :], xbufs[buf_i], sems[buf_i]
            )
        ]

    copies = [fire_chunk(0, 0), None]
    cp_tw.wait()

    lane = lax.iota(jnp.int32, LANES)
    hi_mask = jnp.int32(-65536)
    lo6_mask = jnp.int32(63)
    hi_sel = jnp.int32(64)

    for c in range(NUM_CHUNKS):
        cur = c % 2
        nxt = (c + 1) % 2
        if c + 1 < NUM_CHUNKS:
            copies[nxt] = fire_chunk(c + 1, nxt)
        for cp in copies[cur]:
            cp.wait()
        xb = xbufs[cur]
        for g in range(GROUPS_PER_CHUNK):
            row_idx = g * LANES + lane

            def s_step(i, acc, row_idx=row_idx, xb=xb):
                s0 = i * S_UNROLL
                for k in range(S_UNROLL):
                    col = jnp.full((LANES,), s0 + k, jnp.int32)
                    tok = plsc.load_gather(xb, [row_idx, col])
                    widx = lax.bitwise_or(
                        lax.shift_left(lax.shift_right_logical(tok, 7), 6),
                        lax.bitwise_and(tok, lo6_mask),
                    )
                    pk = plsc.load_gather(twbuf, [widx])
                    in_hi = lax.bitwise_and(tok, hi_sel) != 0
                    bits = jnp.where(
                        in_hi, lax.bitwise_and(pk, hi_mask), lax.shift_left(pk, 16)
                    )
                    acc = acc + plsc.bitcast(bits, jnp.float32)
                return acc

            acc = lax.fori_loop(
                0, SEQ_N // S_UNROLL, s_step, jnp.zeros((LANES,), jnp.float32)
            )
            z = acc * jnp.float32(1.0 / SEQ_N)
            res = 1.0 / (1.0 + jnp.exp(-z))
            outbuf[pl.ds(c * CHUNK_ROWS + g * LANES, LANES)] = res

    pltpu.sync_copy(outbuf, out_hbm.at[pl.ds(row_base, ROWS_PER_WORKER)])


@jax.jit
def kernel(x, table, W, b):
    packed = _compute_packed_tw(table, W, b)
    x2d = x.astype(jnp.int32)

    mesh = plsc.VectorSubcoreMesh(
        core_axis_name="c",
        subcore_axis_name="s",
        num_cores=NUM_CORES,
        num_subcores=NUM_SUBCORES,
    )
    out1d = pl.kernel(
        _sc_body,
        out_type=jax.ShapeDtypeStruct((BATCH_N,), jnp.float32),
        mesh=mesh,
        compiler_params=pltpu.CompilerParams(needs_layout_passes=False),
        scratch_types=[
            pltpu.VMEM((PACKED_WORDS,), jnp.int32),
            pltpu.VMEM((CHUNK_ROWS, SEQ_N), jnp.int32),
            pltpu.VMEM((CHUNK_ROWS, SEQ_N), jnp.int32),
            pltpu.VMEM((ROWS_PER_WORKER,), jnp.float32),
            pltpu.SemaphoreType.DMA,
            pltpu.SemaphoreType.DMA,
            pltpu.SemaphoreType.DMA,
        ],
    )(packed, x2d)
    return out1d.reshape(BATCH_N, 1)


# speedup vs baseline: 4.6725x; 3.2562x over previous
import jax
import jax.numpy as jnp
from jax import lax
from jax.experimental import pallas as pl
from jax.experimental.pallas import tpu as pltpu
from jax.experimental.pallas import tpu_sc as plsc

VOCAB_SIZE = 170000
EMB_D = 16
BATCH_N = 16384
SEQ_N = 200

NUM_CORES = 2
NUM_SUBCORES = 16
NUM_WORKERS = NUM_CORES * NUM_SUBCORES
COLS_PER_WORKER = BATCH_N // NUM_WORKERS
CHUNK_COLS = 128
COL_CHUNKS = COLS_PER_WORKER // CHUNK_COLS
F_ROWS = 104
S_ROWS = SEQ_N - F_ROWS
LANES = 16
GROUPS_PER_CHUNK = CHUNK_COLS // LANES
S_UNROLL = 8

PAIR_OFFSET = 87040
TC_VOCAB_PER_BLOCK = 17408
TC_GRID = 5
TC_CHUNKS = TC_VOCAB_PER_BLOCK // 128
TC_OUT_ROWS = TC_CHUNKS
PACKED_WORDS = TC_GRID * TC_OUT_ROWS * 128
TW_WORDS = PACKED_WORDS


def _bf16_rn_bits(x):
    xb = lax.bitcast_convert_type(x, jnp.int32)
    odd = lax.bitwise_and(lax.shift_right_logical(xb, 16), jnp.int32(1))
    return lax.shift_right_logical(xb + jnp.int32(0x7FFF) + odd, 16)


def _tc_tw_body(tlo_ref, thi_ref, w_ref, b_ref, out_ref):
    w3 = jnp.transpose(w_ref[...], (1, 0)).reshape(EMB_D, 1, 1)

    def tw_bits(ref):
        t3 = ref[...].reshape(EMB_D, TC_CHUNKS, 128)
        return _bf16_rn_bits(jnp.sum(t3 * w3, axis=0) + b_ref[0, 0])

    out_ref[...] = lax.bitwise_or(
        tw_bits(tlo_ref), lax.shift_left(tw_bits(thi_ref), 16)
    )


def _compute_packed_tw(tableT, W, b):
    packed2 = pl.pallas_call(
        _tc_tw_body,
        grid=(TC_GRID,),
        out_shape=jax.ShapeDtypeStruct((TC_GRID * TC_OUT_ROWS, 128), jnp.int32),
        in_specs=[
            pl.BlockSpec((EMB_D, TC_VOCAB_PER_BLOCK), lambda g: (0, g)),
            pl.BlockSpec((EMB_D, TC_VOCAB_PER_BLOCK), lambda g: (0, g + TC_GRID)),
            pl.BlockSpec((1, EMB_D), lambda g: (0, 0)),
            pl.BlockSpec(memory_space=pltpu.SMEM),
        ],
        out_specs=pl.BlockSpec((TC_OUT_ROWS, 128), lambda g: (g, 0)),
    )(tableT, tableT, W.astype(jnp.float32), b.reshape(1, 1).astype(jnp.float32))
    return packed2.reshape(PACKED_WORDS)


def _sc_body(tw_hbm, xT_hbm, out_hbm, twbuf, xf0, xf1, xs, outbuf, sem_tw, sem_f, sem_s):
    wid = lax.axis_index("s") * NUM_CORES + lax.axis_index("c")
    col_base = wid * COLS_PER_WORKER

    cp_tw = pltpu.async_copy(tw_hbm.at[pl.ds(0, TW_WORDS)], twbuf, sem_tw)
    xfs = (xf0, xf1)

    def fire_f(c_idx, par):
        c0 = col_base + c_idx * CHUNK_COLS
        return pltpu.async_copy(
            xT_hbm.at[pl.ds(0, F_ROWS), pl.ds(c0, CHUNK_COLS)], xfs[par], sem_f
        )

    def fire_s(c_idx):
        c0 = col_base + c_idx * CHUNK_COLS
        return pltpu.async_copy(
            xT_hbm.at[pl.ds(F_ROWS, S_ROWS), pl.ds(c0, CHUNK_COLS)], xs, sem_s
        )

    cp_f = fire_f(0, 0)
    cp_s = fire_s(0)
    cp_tw.wait()

    pair_off = jnp.int32(PAIR_OFFSET)
    zero_i = jnp.int32(0)

    def run_half(xb, g, nrows, accs):
        def s_step(i, accs):
            accs = list(accs)
            s0 = i * S_UNROLL
            for k in range(S_UNROLL):
                tok = xb[s0 + k, pl.ds(g * LANES, LANES)]
                in_hi = tok >= pair_off
                widx = tok - jnp.where(in_hi, pair_off, zero_i)
                pk = plsc.load_gather(twbuf, [widx])
                bits = jnp.where(in_hi, pk, lax.shift_left(pk, 16))
                accs[k % 4] = accs[k % 4] + plsc.bitcast(bits, jnp.float32)
            return tuple(accs)

        return plsc.parallel_loop(0, nrows // S_UNROLL, 1, unroll=2, carry=accs)(s_step)

    for c in range(COL_CHUNKS):
        par = c % 2
        cp_f.wait()
        if c + 1 < COL_CHUNKS:
            cp_f = fire_f(c + 1, 1 - par)
        cp_s.wait()
        xf = xfs[par]
        for g in range(GROUPS_PER_CHUNK):
            zero = jnp.zeros((LANES,), jnp.float32)
            accs = run_half(xf, g, F_ROWS, (zero, zero, zero, zero))
            accs = run_half(xs, g, S_ROWS, accs)
            acc = (accs[0] + accs[1]) + (accs[2] + accs[3])
            z = acc * jnp.float32(1.0 / SEQ_N)
            res = 1.0 / (1.0 + jnp.exp(-z))
            outbuf[pl.ds(c * CHUNK_COLS + g * LANES, LANES)] = res
        if c + 1 < COL_CHUNKS:
            cp_s = fire_s(c + 1)

    pltpu.sync_copy(outbuf, out_hbm.at[pl.ds(col_base, COLS_PER_WORKER)])


@jax.jit
def kernel(x, table, W, b):
    packed = _compute_packed_tw(table.T, W, b)
    xT = x.T.astype(jnp.int32)

    mesh = plsc.VectorSubcoreMesh(
        core_axis_name="c",
        subcore_axis_name="s",
        num_cores=NUM_CORES,
        num_subcores=NUM_SUBCORES,
    )
    out1d = pl.kernel(
        _sc_body,
        out_type=jax.ShapeDtypeStruct((BATCH_N,), jnp.float32),
        mesh=mesh,
        compiler_params=pltpu.CompilerParams(needs_layout_passes=False),
        scratch_types=[
            pltpu.VMEM((TW_WORDS,), jnp.int32),
            pltpu.VMEM((F_ROWS, CHUNK_COLS), jnp.int32),
            pltpu.VMEM((F_ROWS, CHUNK_COLS), jnp.int32),
            pltpu.VMEM((S_ROWS, CHUNK_COLS), jnp.int32),
            pltpu.VMEM((COLS_PER_WORKER,), jnp.float32),
            pltpu.SemaphoreType.DMA,
            pltpu.SemaphoreType.DMA,
            pltpu.SemaphoreType.DMA,
        ],
    )(packed, xT)
    return out1d.reshape(BATCH_N, 1)
